# Initial kernel scaffold; baseline (speedup 1.0000x reference)
import functools

import jax
import jax.numpy as jnp
from jax import lax
from jax.experimental import pallas as pl
from jax.experimental.pallas import tpu as pltpu
from jax.experimental.pallas import tpu_sc as plsc

N_NODES = 10000
N_HE = 10000
NNZ = 320000
C = 128
TEMB_C = 512
HALF = 64
W = 80
NC = 2
NS = 16
L = 16
EPS = NNZ // NS
CH = 125
NCH = EPS // CH
RPS = N_NODES // NS
RCH = 125
NRCH = RPS // RCH
RB = 1000
GRID = N_NODES // RB


def _pre_body(x_ref, wc_ref, temb_ref, wt_ref, bt_ref, xaug_ref, t_ref):
    xt = jnp.dot(x_ref[...], wc_ref[...], preferred_element_type=jnp.float32)
    r = xt.shape[0]
    ones = jnp.ones((r, 1), jnp.float32)
    pad = jnp.zeros((r, W - HALF - 1), jnp.float32)
    h0 = jnp.concatenate([xt[:, :HALF], ones, pad], axis=1)
    h1 = jnp.concatenate([xt[:, HALF:], ones, pad], axis=1)
    xaug_ref[...] = jnp.stack([h0, h1], axis=0)
    s = temb_ref[...]
    s = s * jax.nn.sigmoid(s)
    t_ref[...] = jnp.dot(s, wt_ref[...], preferred_element_type=jnp.float32) + bt_ref[...]


def _post_body(oa_ref, t_ref, bh_ref, wp_ref, bp_ref, h_ref):
    o = jnp.concatenate([oa_ref[0, :, :HALF], oa_ref[1, :, :HALF]], axis=1)
    o = o + bh_ref[...]
    hh = jnp.dot(o, wp_ref[...], preferred_element_type=jnp.float32)
    hh = hh + bp_ref[...] + t_ref[...]
    h_ref[...] = hh * jax.nn.sigmoid(hh)


def _sc_body(xflat_hbm, idxo_hbm, idxn_hbm, idxh_hbm, out_hbm,
             idxo_v, idxn_v, idxh_v, rows_v, he_sh, out_sh, sem):
    c = lax.axis_index("c")
    s = lax.axis_index("s")

    pltpu.sync_copy(idxo_hbm.at[c].at[s], idxo_v)
    pltpu.sync_copy(idxn_hbm.at[s], idxn_v)
    pltpu.sync_copy(idxh_hbm.at[s], idxh_v)

    def zrow(i, carry):
        rr = i // (W // L)
        g = i % (W // L)
        rows_v[rr, pl.ds(g * L, L)] = jnp.zeros((L,), jnp.float32)
        return carry
    lax.fori_loop(0, RCH * (W // L), zrow, 0)
    for q in range(NRCH):
        base = s * RPS + q * RCH
        pltpu.sync_copy(rows_v, he_sh.at[pl.ds(base, RCH)])
        pltpu.sync_copy(rows_v, out_sh.at[pl.ds(base, RCH)])
    plsc.subcore_barrier()

    def p1(j, carry):
        pltpu.async_copy(xflat_hbm.at[idxo_v.at[j]], rows_v, sem).wait()
        pltpu.sync_copy(rows_v, he_sh.at[idxh_v.at[j]], add=True)
        return carry
    lax.fori_loop(0, NCH, p1, 0)
    plsc.subcore_barrier()

    def scale_rows(shared_ref, q):
        base = s * RPS + q * RCH
        pltpu.sync_copy(shared_ref.at[pl.ds(base, RCH)], rows_v)

        def srow(r, carry):
            d = rows_v[r, HALF]
            inv = jnp.where(d > 0.0, 1.0 / d, 0.0)
            vinv = jnp.full((L,), inv, jnp.float32)
            for g in range(W // L):
                rows_v[r, pl.ds(g * L, L)] = rows_v[r, pl.ds(g * L, L)] * vinv
            return carry
        lax.fori_loop(0, RCH, srow, 0)

    for q in range(NRCH):
        base = s * RPS + q * RCH
        scale_rows(he_sh, q)
        pltpu.sync_copy(rows_v, he_sh.at[pl.ds(base, RCH)])
    plsc.subcore_barrier()

    def p2(j, carry):
        pltpu.async_copy(he_sh.at[idxh_v.at[j]], rows_v, sem).wait()
        pltpu.sync_copy(rows_v, out_sh.at[idxn_v.at[j]], add=True)
        return carry
    lax.fori_loop(0, NCH, p2, 0)
    plsc.subcore_barrier()

    for q in range(NRCH):
        base = s * RPS + q * RCH
        scale_rows(out_sh, q)
        pltpu.sync_copy(rows_v, out_hbm.at[c].at[pl.ds(base, RCH)])


def _make_calls():
    pre = pl.pallas_call(
        _pre_body,
        grid=(GRID,),
        in_specs=[
            pl.BlockSpec((RB, C), lambda i: (i, 0)),
            pl.BlockSpec((C, C), lambda i: (0, 0)),
            pl.BlockSpec((RB, TEMB_C), lambda i: (i, 0)),
            pl.BlockSpec((TEMB_C, C), lambda i: (0, 0)),
            pl.BlockSpec((1, C), lambda i: (0, 0)),
        ],
        out_specs=[
            pl.BlockSpec((NC, RB, W), lambda i: (0, i, 0)),
            pl.BlockSpec((RB, C), lambda i: (i, 0)),
        ],
        out_shape=[
            jax.ShapeDtypeStruct((NC, N_NODES, W), jnp.float32),
            jax.ShapeDtypeStruct((N_NODES, C), jnp.float32),
        ],
    )
    post = pl.pallas_call(
        _post_body,
        grid=(GRID,),
        in_specs=[
            pl.BlockSpec((NC, RB, W), lambda i: (0, i, 0)),
            pl.BlockSpec((RB, C), lambda i: (i, 0)),
            pl.BlockSpec((1, C), lambda i: (0, 0)),
            pl.BlockSpec((C, C), lambda i: (0, 0)),
            pl.BlockSpec((1, C), lambda i: (0, 0)),
        ],
        out_specs=pl.BlockSpec((RB, C), lambda i: (i, 0)),
        out_shape=jax.ShapeDtypeStruct((N_NODES, C), jnp.float32),
    )
    sc = pl.kernel(
        _sc_body,
        out_type=jax.ShapeDtypeStruct((NC, N_NODES, W), jnp.float32),
        mesh=plsc.VectorSubcoreMesh(
            core_axis_name="c", subcore_axis_name="s",
            num_cores=NC, num_subcores=NS,
        ),
        scratch_types=[
            pltpu.VMEM((NCH, CH), jnp.int32),
            pltpu.VMEM((NCH, CH), jnp.int32),
            pltpu.VMEM((NCH, CH), jnp.int32),
            pltpu.VMEM((RCH, W), jnp.float32),
            pltpu.VMEM_SHARED((N_NODES, W), jnp.float32),
            pltpu.VMEM_SHARED((N_NODES, W), jnp.float32),
            pltpu.SemaphoreType.DMA,
        ],
    )
    return pre, post, sc


def kernel(x, incidence_matrix, temb, W_conv, b_hconv, W_proj, b_proj, W_time, b_time):
    pre, post, sc = _make_calls()
    node_idx = incidence_matrix[0]
    he_idx = incidence_matrix[1]
    xaug, t = pre(x, W_conv, temb, W_time, b_time.reshape(1, C))
    xflat = xaug.reshape(NC * N_NODES, W)
    idxo = jnp.concatenate([node_idx, node_idx + N_NODES]).reshape(NC, NS, NCH, CH)
    idxn = node_idx.reshape(NS, NCH, CH)
    idxh = he_idx.reshape(NS, NCH, CH)
    outaug = sc(xflat, idxo, idxn, idxh)
    h = post(outaug, t, b_hconv.reshape(1, C), W_proj, b_proj.reshape(1, C))
    return (h, jnp.zeros_like(x))


# speedup vs baseline: 15.5815x; 15.5815x over previous
import functools

import jax
import jax.numpy as jnp
from jax import lax
from jax.experimental import pallas as pl
from jax.experimental.pallas import tpu as pltpu
from jax.experimental.pallas import tpu_sc as plsc

N_NODES = 10000
N_HE = 10000
NNZ = 320000
C = 128
TEMB_C = 512
HALF = 64
W = 80
NC = 2
NS = 16
L = 16
EPS = NNZ // NS
CH = 125
NCH = EPS // CH
N_PAD = 10240
RPS = N_PAD // NS
RCH = 128
NRCH = RPS // RCH
RB = 1000
GRID = N_NODES // RB


def _pre_body(x_ref, wc_ref, temb_ref, wt_ref, bt_ref, xaug_ref, t_ref):
    xt = jnp.dot(x_ref[...], wc_ref[...], preferred_element_type=jnp.float32)
    r = xt.shape[0]
    ones = jnp.ones((r, 1), jnp.float32)
    pad = jnp.zeros((r, W - HALF - 1), jnp.float32)
    h0 = jnp.concatenate([xt[:, :HALF], ones, pad], axis=1)
    h1 = jnp.concatenate([xt[:, HALF:], ones, pad], axis=1)
    xaug_ref[...] = jnp.stack([h0, h1], axis=0)
    s = temb_ref[...]
    s = s * jax.nn.sigmoid(s)
    t_ref[...] = jnp.dot(s, wt_ref[...], preferred_element_type=jnp.float32) + bt_ref[...]


def _post_body(oa_ref, t_ref, bh_ref, wp_ref, bp_ref, h_ref):
    o = jnp.concatenate([oa_ref[0, :, :HALF], oa_ref[1, :, :HALF]], axis=1)
    o = o + bh_ref[...]
    hh = jnp.dot(o, wp_ref[...], preferred_element_type=jnp.float32)
    hh = hh + bp_ref[...] + t_ref[...]
    h_ref[...] = hh * jax.nn.sigmoid(hh)


def _sc_body(xaug_hbm, idxn_hbm, idxh_hbm,
             out_hbm, he_hbm,
             idxn_v, idxh_v, grows_v, srows_v, acc_sh, sem):
    c = lax.axis_index("c")
    s = lax.axis_index("s")

    pltpu.sync_copy(idxn_hbm.at[s], idxn_v)
    pltpu.sync_copy(idxh_hbm.at[s], idxh_v)

    def zero_srows():
        def zrow(i, carry):
            rr = i // (W // L)
            g = i % (W // L)
            srows_v[rr, pl.ds(g * L, L)] = jnp.zeros((L,), jnp.float32)
            return carry
        lax.fori_loop(0, RCH * (W // L), zrow, 0)

    def zero_acc():
        for q in range(NRCH):
            base = s * RPS + q * RCH
            pltpu.sync_copy(srows_v, acc_sh.at[pl.ds(base, RCH)])

    def scale_srows():
        def srow(r, carry):
            dvec = srows_v[r, pl.ds(HALF, L)]
            invvec = jnp.where(dvec > 0.0, 1.0 / dvec, 0.0)
            vinv = jnp.full((L,), invvec[0], jnp.float32)
            for g in range(W // L):
                srows_v[r, pl.ds(g * L, L)] = srows_v[r, pl.ds(g * L, L)] * vinv
            return carry
        lax.fori_loop(0, RCH, srow, 0)

    zero_srows()
    zero_acc()
    plsc.subcore_barrier()

    def p1(j, carry):
        pltpu.async_copy(xaug_hbm.at[c].at[idxn_v.at[j]], grows_v, sem).wait()
        pltpu.sync_copy(grows_v, acc_sh.at[idxh_v.at[j]], add=True)
        return carry
    lax.fori_loop(0, NCH, p1, 0)
    plsc.subcore_barrier()

    for q in range(NRCH):
        base = s * RPS + q * RCH
        pltpu.sync_copy(acc_sh.at[pl.ds(base, RCH)], srows_v)
        scale_srows()
        pltpu.sync_copy(srows_v, he_hbm.at[c].at[pl.ds(base, RCH)])
    zero_srows()
    zero_acc()
    plsc.subcore_barrier()

    def p2(j, carry):
        pltpu.async_copy(he_hbm.at[c].at[idxh_v.at[j]], grows_v, sem).wait()
        pltpu.sync_copy(grows_v, acc_sh.at[idxn_v.at[j]], add=True)
        return carry
    lax.fori_loop(0, NCH, p2, 0)
    plsc.subcore_barrier()

    for q in range(NRCH):
        base = s * RPS + q * RCH
        pltpu.sync_copy(acc_sh.at[pl.ds(base, RCH)], srows_v)
        scale_srows()
        pltpu.sync_copy(srows_v, out_hbm.at[c].at[pl.ds(base, RCH)])


def _make_calls():
    pre = pl.pallas_call(
        _pre_body,
        grid=(GRID,),
        in_specs=[
            pl.BlockSpec((RB, C), lambda i: (i, 0)),
            pl.BlockSpec((C, C), lambda i: (0, 0)),
            pl.BlockSpec((RB, TEMB_C), lambda i: (i, 0)),
            pl.BlockSpec((TEMB_C, C), lambda i: (0, 0)),
            pl.BlockSpec((1, C), lambda i: (0, 0)),
        ],
        out_specs=[
            pl.BlockSpec((NC, RB, W), lambda i: (0, i, 0)),
            pl.BlockSpec((RB, C), lambda i: (i, 0)),
        ],
        out_shape=[
            jax.ShapeDtypeStruct((NC, N_NODES, W), jnp.float32),
            jax.ShapeDtypeStruct((N_NODES, C), jnp.float32),
        ],
    )
    post = pl.pallas_call(
        _post_body,
        grid=(GRID,),
        in_specs=[
            pl.BlockSpec((NC, RB, W), lambda i: (0, i, 0)),
            pl.BlockSpec((RB, C), lambda i: (i, 0)),
            pl.BlockSpec((1, C), lambda i: (0, 0)),
            pl.BlockSpec((C, C), lambda i: (0, 0)),
            pl.BlockSpec((1, C), lambda i: (0, 0)),
        ],
        out_specs=pl.BlockSpec((RB, C), lambda i: (i, 0)),
        out_shape=jax.ShapeDtypeStruct((N_NODES, C), jnp.float32),
    )
    sc = pl.kernel(
        _sc_body,
        out_type=[
            jax.ShapeDtypeStruct((NC, N_PAD, W), jnp.float32),
            jax.ShapeDtypeStruct((NC, N_PAD, W), jnp.float32),
        ],
        mesh=plsc.VectorSubcoreMesh(
            core_axis_name="c", subcore_axis_name="s",
            num_cores=NC, num_subcores=NS,
        ),
        scratch_types=[
            pltpu.VMEM((NCH, CH), jnp.int32),
            pltpu.VMEM((NCH, CH), jnp.int32),
            pltpu.VMEM((CH, W), jnp.float32),
            pltpu.VMEM((RCH, W), jnp.float32),
            pltpu.VMEM_SHARED((N_PAD, W), jnp.float32),
            pltpu.SemaphoreType.DMA,
        ],
        compiler_params=pltpu.CompilerParams(use_tc_tiling_on_sc=False),
    )
    return pre, post, sc


def kernel(x, incidence_matrix, temb, W_conv, b_hconv, W_proj, b_proj, W_time, b_time):
    pre, post, sc = _make_calls()
    node_idx = incidence_matrix[0]
    he_idx = incidence_matrix[1]
    xaug, t = pre(x, W_conv, temb, W_time, b_time.reshape(1, C))
    idxn = node_idx.reshape(NS, NCH, CH)
    idxh = he_idx.reshape(NS, NCH, CH)
    outpad, _he = sc(xaug, idxn, idxh)
    outaug = outpad[:, :N_NODES, :]
    h = post(outaug, t, b_hconv.reshape(1, C), W_proj, b_proj.reshape(1, C))
    return (h, jnp.zeros_like(x))


# speedup vs baseline: 19.6846x; 1.2633x over previous
import functools

import jax
import jax.numpy as jnp
from jax import lax
from jax.experimental import pallas as pl
from jax.experimental.pallas import tpu as pltpu
from jax.experimental.pallas import tpu_sc as plsc

N_NODES = 10000
N_HE = 10000
NNZ = 320000
C = 128
TEMB_C = 512
HALF = 64
W = 80
NC = 2
NS = 16
L = 16
EPS = NNZ // NS
CH = 125
NCH = EPS // CH
N_PAD = 10240
RPS = N_PAD // NS
RCH = 128
NRCH = RPS // RCH
RB = 1000
GRID = N_NODES // RB


def _pre_body(x_ref, wc_ref, temb_ref, wt_ref, bt_ref, xaug_ref, t_ref):
    xt = jnp.dot(x_ref[...], wc_ref[...], preferred_element_type=jnp.float32)
    r = xt.shape[0]
    ones = jnp.ones((r, 1), jnp.float32)
    pad = jnp.zeros((r, W - HALF - 1), jnp.float32)
    h0 = jnp.concatenate([xt[:, :HALF], ones, pad], axis=1)
    h1 = jnp.concatenate([xt[:, HALF:], ones, pad], axis=1)
    xaug_ref[...] = jnp.stack([h0, h1], axis=0)
    s = temb_ref[...]
    s = s * jax.nn.sigmoid(s)
    t_ref[...] = jnp.dot(s, wt_ref[...], preferred_element_type=jnp.float32) + bt_ref[...]


def _post_body(oa_ref, t_ref, bh_ref, wp_ref, bp_ref, h_ref):
    o = jnp.concatenate([oa_ref[0, :, :HALF], oa_ref[1, :, :HALF]], axis=1)
    o = o + bh_ref[...]
    hh = jnp.dot(o, wp_ref[...], preferred_element_type=jnp.float32)
    hh = hh + bp_ref[...] + t_ref[...]
    h_ref[...] = hh * jax.nn.sigmoid(hh)


def _sc_body(xaug_hbm, idxn_hbm, idxh_hbm,
             out_hbm, he_hbm,
             idxn_v, idxh_v, grows_v, srows_v, acc_sh, sem0, sem1):
    c = lax.axis_index("c")
    s = lax.axis_index("s")

    pltpu.sync_copy(idxn_hbm.at[s], idxn_v)
    pltpu.sync_copy(idxh_hbm.at[s], idxh_v)

    def zero_srows():
        def zrow(i, carry):
            rr = i // (W // L)
            g = i % (W // L)
            srows_v[rr, pl.ds(g * L, L)] = jnp.zeros((L,), jnp.float32)
            return carry
        lax.fori_loop(0, RCH * (W // L), zrow, 0)

    def zero_acc():
        for q in range(NRCH):
            base = s * RPS + q * RCH
            pltpu.sync_copy(srows_v, acc_sh.at[pl.ds(base, RCH)])

    def scale_srows():
        def srow(r, carry):
            dvec = srows_v[r, pl.ds(HALF, L)]
            invvec = jnp.where(dvec > 0.0, 1.0 / dvec, 0.0)
            vinv = jnp.full((L,), invvec[0], jnp.float32)
            for g in range(W // L):
                srows_v[r, pl.ds(g * L, L)] = srows_v[r, pl.ds(g * L, L)] * vinv
            return carry
        lax.fori_loop(0, RCH, srow, 0)

    zero_srows()
    zero_acc()
    plsc.subcore_barrier()

    def run_pass(gather_from, idxg_v, idxs_v):
        def start_gather(j, buf, sm):
            pltpu.async_copy(gather_from.at[idxg_v.at[j]], buf, sm)

        start_gather(0, grows_v.at[0], sem0)

        def body(k, carry):
            j0 = 2 * k
            j1 = 2 * k + 1
            j2 = lax.rem(2 * k + 2, NCH)
            pltpu.make_async_copy(gather_from.at[idxg_v.at[j0]],
                                  grows_v.at[0], sem0).wait()
            start_gather(j1, grows_v.at[1], sem1)
            pltpu.sync_copy(grows_v.at[0], acc_sh.at[idxs_v.at[j0]], add=True)
            pltpu.make_async_copy(gather_from.at[idxg_v.at[j1]],
                                  grows_v.at[1], sem1).wait()
            start_gather(j2, grows_v.at[0], sem0)
            pltpu.sync_copy(grows_v.at[1], acc_sh.at[idxs_v.at[j1]], add=True)
            return carry
        lax.fori_loop(0, NCH // 2, body, 0)
        pltpu.make_async_copy(gather_from.at[idxg_v.at[0]],
                              grows_v.at[0], sem0).wait()

    run_pass(xaug_hbm.at[c], idxn_v, idxh_v)
    plsc.subcore_barrier()

    for q in range(NRCH):
        base = s * RPS + q * RCH
        pltpu.sync_copy(acc_sh.at[pl.ds(base, RCH)], srows_v)
        scale_srows()
        pltpu.sync_copy(srows_v, he_hbm.at[c].at[pl.ds(base, RCH)])
    zero_srows()
    zero_acc()
    plsc.subcore_barrier()

    run_pass(he_hbm.at[c], idxh_v, idxn_v)
    plsc.subcore_barrier()

    for q in range(NRCH):
        base = s * RPS + q * RCH
        pltpu.sync_copy(acc_sh.at[pl.ds(base, RCH)], srows_v)
        scale_srows()
        pltpu.sync_copy(srows_v, out_hbm.at[c].at[pl.ds(base, RCH)])


def _make_calls():
    pre = pl.pallas_call(
        _pre_body,
        grid=(GRID,),
        in_specs=[
            pl.BlockSpec((RB, C), lambda i: (i, 0)),
            pl.BlockSpec((C, C), lambda i: (0, 0)),
            pl.BlockSpec((RB, TEMB_C), lambda i: (i, 0)),
            pl.BlockSpec((TEMB_C, C), lambda i: (0, 0)),
            pl.BlockSpec((1, C), lambda i: (0, 0)),
        ],
        out_specs=[
            pl.BlockSpec((NC, RB, W), lambda i: (0, i, 0)),
            pl.BlockSpec((RB, C), lambda i: (i, 0)),
        ],
        out_shape=[
            jax.ShapeDtypeStruct((NC, N_NODES, W), jnp.float32),
            jax.ShapeDtypeStruct((N_NODES, C), jnp.float32),
        ],
    )
    post = pl.pallas_call(
        _post_body,
        grid=(GRID,),
        in_specs=[
            pl.BlockSpec((NC, RB, W), lambda i: (0, i, 0)),
            pl.BlockSpec((RB, C), lambda i: (i, 0)),
            pl.BlockSpec((1, C), lambda i: (0, 0)),
            pl.BlockSpec((C, C), lambda i: (0, 0)),
            pl.BlockSpec((1, C), lambda i: (0, 0)),
        ],
        out_specs=pl.BlockSpec((RB, C), lambda i: (i, 0)),
        out_shape=jax.ShapeDtypeStruct((N_NODES, C), jnp.float32),
    )
    sc = pl.kernel(
        _sc_body,
        out_type=[
            jax.ShapeDtypeStruct((NC, N_PAD, W), jnp.float32),
            jax.ShapeDtypeStruct((NC, N_PAD, W), jnp.float32),
        ],
        mesh=plsc.VectorSubcoreMesh(
            core_axis_name="c", subcore_axis_name="s",
            num_cores=NC, num_subcores=NS,
        ),
        scratch_types=[
            pltpu.VMEM((NCH, CH), jnp.int32),
            pltpu.VMEM((NCH, CH), jnp.int32),
            pltpu.VMEM((2, CH, W), jnp.float32),
            pltpu.VMEM((RCH, W), jnp.float32),
            pltpu.VMEM_SHARED((N_PAD, W), jnp.float32),
            pltpu.SemaphoreType.DMA,
            pltpu.SemaphoreType.DMA,
        ],
        compiler_params=pltpu.CompilerParams(use_tc_tiling_on_sc=False),
    )
    return pre, post, sc


def kernel(x, incidence_matrix, temb, W_conv, b_hconv, W_proj, b_proj, W_time, b_time):
    pre, post, sc = _make_calls()
    node_idx = incidence_matrix[0]
    he_idx = incidence_matrix[1]
    xaug, t = pre(x, W_conv, temb, W_time, b_time.reshape(1, C))
    idxn = node_idx.reshape(NS, NCH, CH)
    idxh = he_idx.reshape(NS, NCH, CH)
    outpad, _he = sc(xaug, idxn, idxh)
    outaug = outpad[:, :N_NODES, :]
    h = post(outaug, t, b_hconv.reshape(1, C), W_proj, b_proj.reshape(1, C))
    return (h, jnp.zeros_like(x))


# speedup vs baseline: 20.3104x; 1.0318x over previous
import functools

import jax
import jax.numpy as jnp
from jax import lax
from jax.experimental import pallas as pl
from jax.experimental.pallas import tpu as pltpu
from jax.experimental.pallas import tpu_sc as plsc

N_NODES = 10000
N_HE = 10000
NNZ = 320000
C = 128
TEMB_C = 512
HALF = 64
W = 80
NC = 2
NS = 16
L = 16
EPS = NNZ // NS
CH = 125
NCH = EPS // CH
N_PAD = 10240
RPS = N_PAD // NS
RCH = 128
NRCH = RPS // RCH
RB = 1000
GRID = N_NODES // RB


def _pre_body(x_ref, wc_ref, temb_ref, wt_ref, bt_ref, xaug_ref, t_ref):
    xt = jnp.dot(x_ref[...], wc_ref[...], preferred_element_type=jnp.float32)
    r = xt.shape[0]
    ones = jnp.ones((r, 1), jnp.float32)
    pad = jnp.zeros((r, W - HALF - 1), jnp.float32)
    h0 = jnp.concatenate([xt[:, :HALF], ones, pad], axis=1)
    h1 = jnp.concatenate([xt[:, HALF:], ones, pad], axis=1)
    xaug_ref[...] = jnp.stack([h0, h1], axis=0)
    s = temb_ref[...]
    s = s * jax.nn.sigmoid(s)
    t_ref[...] = jnp.dot(s, wt_ref[...], preferred_element_type=jnp.float32) + bt_ref[...]


def _post_body(oa_ref, t_ref, bh_ref, wp_ref, bp_ref, h_ref):
    o = jnp.concatenate([oa_ref[0, :, :HALF], oa_ref[1, :, :HALF]], axis=1)
    o = o + bh_ref[...]
    hh = jnp.dot(o, wp_ref[...], preferred_element_type=jnp.float32)
    hh = hh + bp_ref[...] + t_ref[...]
    h_ref[...] = hh * jax.nn.sigmoid(hh)


def _sc_body(xaug_hbm, idxn_hbm, idxh_hbm,
             out_hbm, he_hbm,
             idxn_v, idxh_v, grows_v, srows_v, acc_sh, gsem0, gsem1, ssem0, ssem1):
    c = lax.axis_index("c")
    s = lax.axis_index("s")

    pltpu.sync_copy(idxn_hbm.at[s], idxn_v)
    pltpu.sync_copy(idxh_hbm.at[s], idxh_v)

    def zero_srows():
        def zrow(i, carry):
            rr = i // (W // L)
            g = i % (W // L)
            srows_v[rr, pl.ds(g * L, L)] = jnp.zeros((L,), jnp.float32)
            return carry
        lax.fori_loop(0, RCH * (W // L), zrow, 0)

    def zero_acc():
        for q in range(NRCH):
            base = s * RPS + q * RCH
            pltpu.sync_copy(srows_v, acc_sh.at[pl.ds(base, RCH)])

    def scale_srows():
        def srow(r, carry):
            dvec = srows_v[r, pl.ds(HALF, L)]
            invvec = jnp.where(dvec > 0.0, 1.0 / dvec, 0.0)
            vinv = jnp.full((L,), invvec[0], jnp.float32)
            for g in range(W // L):
                srows_v[r, pl.ds(g * L, L)] = srows_v[r, pl.ds(g * L, L)] * vinv
            return carry
        lax.fori_loop(0, RCH, srow, 0)

    zero_srows()
    zero_acc()
    plsc.subcore_barrier()

    def run_pass(gather_from, idxg_v, idxs_v):
        def g_start(j, b, sm):
            pltpu.async_copy(gather_from.at[idxg_v.at[j]], grows_v.at[b], sm)

        def g_wait(j, b, sm):
            pltpu.make_async_copy(gather_from.at[idxg_v.at[j]],
                                  grows_v.at[b], sm).wait()

        def s_start(j, b, sm):
            pltpu.async_copy(grows_v.at[b], acc_sh.at[idxs_v.at[j]], sm, add=True)

        def s_wait(j, b, sm):
            pltpu.make_async_copy(grows_v.at[b],
                                  acc_sh.at[idxs_v.at[j]], sm).wait()

        g_start(0, 0, gsem0)
        g_start(1, 1, gsem1)

        def body(k, carry):
            j0 = 2 * k
            j1 = 2 * k + 1
            j2 = lax.rem(2 * k + 2, NCH)
            j3 = lax.rem(2 * k + 3, NCH)
            g_wait(j0, 0, gsem0)
            s_start(j0, 0, ssem0)
            g_wait(j1, 1, gsem1)
            s_start(j1, 1, ssem1)
            s_wait(j0, 0, ssem0)
            g_start(j2, 0, gsem0)
            s_wait(j1, 1, ssem1)
            g_start(j3, 1, gsem1)
            return carry
        lax.fori_loop(0, NCH // 2, body, 0)
        g_wait(0, 0, gsem0)
        g_wait(1, 1, gsem1)

    run_pass(xaug_hbm.at[c], idxn_v, idxh_v)
    plsc.subcore_barrier()

    for q in range(NRCH):
        base = s * RPS + q * RCH
        pltpu.sync_copy(acc_sh.at[pl.ds(base, RCH)], srows_v)
        scale_srows()
        pltpu.sync_copy(srows_v, he_hbm.at[c].at[pl.ds(base, RCH)])
    zero_srows()
    zero_acc()
    plsc.subcore_barrier()

    run_pass(he_hbm.at[c], idxh_v, idxn_v)
    plsc.subcore_barrier()

    for q in range(NRCH):
        base = s * RPS + q * RCH
        pltpu.sync_copy(acc_sh.at[pl.ds(base, RCH)], srows_v)
        scale_srows()
        pltpu.sync_copy(srows_v, out_hbm.at[c].at[pl.ds(base, RCH)])


def _make_calls():
    pre = pl.pallas_call(
        _pre_body,
        grid=(GRID,),
        in_specs=[
            pl.BlockSpec((RB, C), lambda i: (i, 0)),
            pl.BlockSpec((C, C), lambda i: (0, 0)),
            pl.BlockSpec((RB, TEMB_C), lambda i: (i, 0)),
            pl.BlockSpec((TEMB_C, C), lambda i: (0, 0)),
            pl.BlockSpec((1, C), lambda i: (0, 0)),
        ],
        out_specs=[
            pl.BlockSpec((NC, RB, W), lambda i: (0, i, 0)),
            pl.BlockSpec((RB, C), lambda i: (i, 0)),
        ],
        out_shape=[
            jax.ShapeDtypeStruct((NC, N_NODES, W), jnp.float32),
            jax.ShapeDtypeStruct((N_NODES, C), jnp.float32),
        ],
    )
    post = pl.pallas_call(
        _post_body,
        grid=(GRID,),
        in_specs=[
            pl.BlockSpec((NC, RB, W), lambda i: (0, i, 0)),
            pl.BlockSpec((RB, C), lambda i: (i, 0)),
            pl.BlockSpec((1, C), lambda i: (0, 0)),
            pl.BlockSpec((C, C), lambda i: (0, 0)),
            pl.BlockSpec((1, C), lambda i: (0, 0)),
        ],
        out_specs=pl.BlockSpec((RB, C), lambda i: (i, 0)),
        out_shape=jax.ShapeDtypeStruct((N_NODES, C), jnp.float32),
    )
    sc = pl.kernel(
        _sc_body,
        out_type=[
            jax.ShapeDtypeStruct((NC, N_PAD, W), jnp.float32),
            jax.ShapeDtypeStruct((NC, N_PAD, W), jnp.float32),
        ],
        mesh=plsc.VectorSubcoreMesh(
            core_axis_name="c", subcore_axis_name="s",
            num_cores=NC, num_subcores=NS,
        ),
        scratch_types=[
            pltpu.VMEM((NCH, CH), jnp.int32),
            pltpu.VMEM((NCH, CH), jnp.int32),
            pltpu.VMEM((2, CH, W), jnp.float32),
            pltpu.VMEM((RCH, W), jnp.float32),
            pltpu.VMEM_SHARED((N_PAD, W), jnp.float32),
            pltpu.SemaphoreType.DMA,
            pltpu.SemaphoreType.DMA,
            pltpu.SemaphoreType.DMA,
            pltpu.SemaphoreType.DMA,
        ],
        compiler_params=pltpu.CompilerParams(use_tc_tiling_on_sc=False),
    )
    return pre, post, sc


def kernel(x, incidence_matrix, temb, W_conv, b_hconv, W_proj, b_proj, W_time, b_time):
    pre, post, sc = _make_calls()
    node_idx = incidence_matrix[0]
    he_idx = incidence_matrix[1]
    xaug, t = pre(x, W_conv, temb, W_time, b_time.reshape(1, C))
    idxn = node_idx.reshape(NS, NCH, CH)
    idxh = he_idx.reshape(NS, NCH, CH)
    outpad, _he = sc(xaug, idxn, idxh)
    h = post(outpad, t, b_hconv.reshape(1, C), W_proj, b_proj.reshape(1, C))
    return (h, jnp.zeros_like(x))


# speedup vs baseline: 21.6113x; 1.0641x over previous
import functools

import jax
import jax.numpy as jnp
from jax import lax
from jax.experimental import pallas as pl
from jax.experimental.pallas import tpu as pltpu
from jax.experimental.pallas import tpu_sc as plsc

N_NODES = 10000
N_HE = 10000
NNZ = 320000
C = 128
TEMB_C = 512
HALF = 64
W = 72
NC = 2
NS = 16
L = 16
EPS = NNZ // NS
CH = 125
NCH = EPS // CH
N_PAD = 10240
RPS = N_PAD // NS
RCH = 128
NRCH = RPS // RCH
RB = 1000
GRID = N_NODES // RB


def _pre_body(x_ref, wc_ref, temb_ref, wt_ref, bt_ref, xaug_ref, t_ref):
    xt = jnp.dot(x_ref[...], wc_ref[...], preferred_element_type=jnp.float32)
    r = xt.shape[0]
    ones = jnp.ones((r, 1), jnp.float32)
    pad = jnp.zeros((r, W - HALF - 1), jnp.float32)
    h0 = jnp.concatenate([xt[:, :HALF], ones, pad], axis=1)
    h1 = jnp.concatenate([xt[:, HALF:], ones, pad], axis=1)
    xaug_ref[...] = jnp.stack([h0, h1], axis=0)
    s = temb_ref[...]
    s = s * jax.nn.sigmoid(s)
    t_ref[...] = jnp.dot(s, wt_ref[...], preferred_element_type=jnp.float32) + bt_ref[...]


def _post_body(oa_ref, t_ref, bh_ref, wp_ref, bp_ref, h_ref):
    o = jnp.concatenate([oa_ref[0, :, :HALF], oa_ref[1, :, :HALF]], axis=1)
    o = o + bh_ref[...]
    hh = jnp.dot(o, wp_ref[...], preferred_element_type=jnp.float32)
    hh = hh + bp_ref[...] + t_ref[...]
    h_ref[...] = hh * jax.nn.sigmoid(hh)


def _sc_body(xaug_hbm, idxn_hbm, idxh_hbm,
             out_hbm, he_hbm,
             idxn_v, idxh_v, grows_v, srows_v, acc_sh, gsem0, gsem1, ssem0, ssem1):
    c = lax.axis_index("c")
    s = lax.axis_index("s")

    pltpu.sync_copy(idxn_hbm.at[s], idxn_v)
    pltpu.sync_copy(idxh_hbm.at[s], idxh_v)

    def zero_srows():
        zv = jnp.zeros((L,), jnp.float32)

        def zrow(rr, carry):
            for off in (0, 16, 32, 48, 56):
                srows_v[rr, pl.ds(off, L)] = zv
            return carry
        lax.fori_loop(0, RCH, zrow, 0)

    def zero_acc():
        for q in range(NRCH):
            base = s * RPS + q * RCH
            pltpu.sync_copy(srows_v, acc_sh.at[pl.ds(base, RCH)])

    def scale_srows():
        def srow(r, carry):
            tail = srows_v[r, pl.ds(56, L)]
            invvec = jnp.where(tail > 0.0, 1.0 / tail, 0.0)
            vinv = jnp.full((L,), invvec[8], jnp.float32)
            for g in range(4):
                srows_v[r, pl.ds(g * L, L)] = srows_v[r, pl.ds(g * L, L)] * vinv
            srows_v[r, pl.ds(56, L)] = tail * vinv
            return carry
        lax.fori_loop(0, RCH, srow, 0)

    zero_srows()
    zero_acc()
    plsc.subcore_barrier()

    def run_pass(gather_from, idxg_v, idxs_v):
        def g_start(j, b, sm):
            pltpu.async_copy(gather_from.at[idxg_v.at[j]], grows_v.at[b], sm)

        def g_wait(j, b, sm):
            pltpu.make_async_copy(gather_from.at[idxg_v.at[j]],
                                  grows_v.at[b], sm).wait()

        def s_start(j, b, sm):
            pltpu.async_copy(grows_v.at[b], acc_sh.at[idxs_v.at[j]], sm, add=True)

        def s_wait(j, b, sm):
            pltpu.make_async_copy(grows_v.at[b],
                                  acc_sh.at[idxs_v.at[j]], sm).wait()

        g_start(0, 0, gsem0)
        g_start(1, 1, gsem1)

        def body(k, carry):
            j0 = 2 * k
            j1 = 2 * k + 1
            j2 = lax.rem(2 * k + 2, NCH)
            j3 = lax.rem(2 * k + 3, NCH)
            g_wait(j0, 0, gsem0)
            s_start(j0, 0, ssem0)
            g_wait(j1, 1, gsem1)
            s_start(j1, 1, ssem1)
            s_wait(j0, 0, ssem0)
            g_start(j2, 0, gsem0)
            s_wait(j1, 1, ssem1)
            g_start(j3, 1, gsem1)
            return carry
        lax.fori_loop(0, NCH // 2, body, 0)
        g_wait(0, 0, gsem0)
        g_wait(1, 1, gsem1)

    run_pass(xaug_hbm.at[c], idxn_v, idxh_v)
    plsc.subcore_barrier()

    for q in range(NRCH):
        base = s * RPS + q * RCH
        pltpu.sync_copy(acc_sh.at[pl.ds(base, RCH)], srows_v)
        scale_srows()
        pltpu.sync_copy(srows_v, he_hbm.at[c].at[pl.ds(base, RCH)])
    zero_srows()
    zero_acc()
    plsc.subcore_barrier()

    run_pass(he_hbm.at[c], idxh_v, idxn_v)
    plsc.subcore_barrier()

    for q in range(NRCH):
        base = s * RPS + q * RCH
        pltpu.sync_copy(acc_sh.at[pl.ds(base, RCH)], srows_v)
        scale_srows()
        pltpu.sync_copy(srows_v, out_hbm.at[c].at[pl.ds(base, RCH)])


def _make_calls():
    pre = pl.pallas_call(
        _pre_body,
        grid=(GRID,),
        in_specs=[
            pl.BlockSpec((RB, C), lambda i: (i, 0)),
            pl.BlockSpec((C, C), lambda i: (0, 0)),
            pl.BlockSpec((RB, TEMB_C), lambda i: (i, 0)),
            pl.BlockSpec((TEMB_C, C), lambda i: (0, 0)),
            pl.BlockSpec((1, C), lambda i: (0, 0)),
        ],
        out_specs=[
            pl.BlockSpec((NC, RB, W), lambda i: (0, i, 0)),
            pl.BlockSpec((RB, C), lambda i: (i, 0)),
        ],
        out_shape=[
            jax.ShapeDtypeStruct((NC, N_NODES, W), jnp.float32),
            jax.ShapeDtypeStruct((N_NODES, C), jnp.float32),
        ],
    )
    post = pl.pallas_call(
        _post_body,
        grid=(GRID,),
        in_specs=[
            pl.BlockSpec((NC, RB, W), lambda i: (0, i, 0)),
            pl.BlockSpec((RB, C), lambda i: (i, 0)),
            pl.BlockSpec((1, C), lambda i: (0, 0)),
            pl.BlockSpec((C, C), lambda i: (0, 0)),
            pl.BlockSpec((1, C), lambda i: (0, 0)),
        ],
        out_specs=pl.BlockSpec((RB, C), lambda i: (i, 0)),
        out_shape=jax.ShapeDtypeStruct((N_NODES, C), jnp.float32),
    )
    sc = pl.kernel(
        _sc_body,
        out_type=[
            jax.ShapeDtypeStruct((NC, N_PAD, W), jnp.float32),
            jax.ShapeDtypeStruct((NC, N_PAD, W), jnp.float32),
        ],
        mesh=plsc.VectorSubcoreMesh(
            core_axis_name="c", subcore_axis_name="s",
            num_cores=NC, num_subcores=NS,
        ),
        scratch_types=[
            pltpu.VMEM((NCH, CH), jnp.int32),
            pltpu.VMEM((NCH, CH), jnp.int32),
            pltpu.VMEM((2, CH, W), jnp.float32),
            pltpu.VMEM((RCH, W), jnp.float32),
            pltpu.VMEM_SHARED((N_PAD, W), jnp.float32),
            pltpu.SemaphoreType.DMA,
            pltpu.SemaphoreType.DMA,
            pltpu.SemaphoreType.DMA,
            pltpu.SemaphoreType.DMA,
        ],
        compiler_params=pltpu.CompilerParams(use_tc_tiling_on_sc=False),
    )
    return pre, post, sc


def kernel(x, incidence_matrix, temb, W_conv, b_hconv, W_proj, b_proj, W_time, b_time):
    pre, post, sc = _make_calls()
    node_idx = incidence_matrix[0]
    he_idx = incidence_matrix[1]
    xaug, t = pre(x, W_conv, temb, W_time, b_time.reshape(1, C))
    idxn = node_idx.reshape(NS, NCH, CH)
    idxh = he_idx.reshape(NS, NCH, CH)
    outpad, _he = sc(xaug, idxn, idxh)
    h = post(outpad, t, b_hconv.reshape(1, C), W_proj, b_proj.reshape(1, C))
    return (h, jnp.zeros_like(x))


# speedup vs baseline: 21.8628x; 1.0116x over previous
import functools

import jax
import jax.numpy as jnp
from jax import lax
from jax.experimental import pallas as pl
from jax.experimental.pallas import tpu as pltpu
from jax.experimental.pallas import tpu_sc as plsc

N_NODES = 10000
N_HE = 10000
NNZ = 320000
C = 128
TEMB_C = 512
HALF = 64
W = 72
NC = 2
NS = 16
L = 16
EPS = NNZ // NS
CH = 125
NCH = EPS // CH
N_PAD = 10240
RPS = N_PAD // NS
RCH = 128
NRCH = RPS // RCH
RB = 1000
GRID = N_NODES // RB


def _pre_x_body(x_ref, wc_ref, xaug_ref):
    xt = jnp.dot(x_ref[...], wc_ref[...], preferred_element_type=jnp.float32)
    r = xt.shape[0]
    ones = jnp.ones((r, 1), jnp.float32)
    pad = jnp.zeros((r, W - HALF - 1), jnp.float32)
    h0 = jnp.concatenate([xt[:, :HALF], ones, pad], axis=1)
    h1 = jnp.concatenate([xt[:, HALF:], ones, pad], axis=1)
    xaug_ref[...] = jnp.stack([h0, h1], axis=0)


def _pre_t_body(temb_ref, wt_ref, bt_ref, t_ref):
    s = temb_ref[...]
    s = s * jax.nn.sigmoid(s)
    t_ref[...] = jnp.dot(s, wt_ref[...], preferred_element_type=jnp.float32) + bt_ref[...]


def _post_body(oa_ref, t_ref, bh_ref, wp_ref, bp_ref, h_ref):
    o = jnp.concatenate([oa_ref[0, :, :HALF], oa_ref[1, :, :HALF]], axis=1)
    o = o + bh_ref[...]
    hh = jnp.dot(o, wp_ref[...], preferred_element_type=jnp.float32)
    hh = hh + bp_ref[...] + t_ref[...]
    h_ref[...] = hh * jax.nn.sigmoid(hh)


def _sc_body(xaug_hbm, idxn_hbm, idxh_hbm,
             out_hbm, he_hbm,
             idxn_v, idxh_v, grows_v, srows_v, acc_sh, gsem0, gsem1, ssem0, ssem1):
    c = lax.axis_index("c")
    s = lax.axis_index("s")

    pltpu.sync_copy(idxn_hbm.at[s], idxn_v)
    pltpu.sync_copy(idxh_hbm.at[s], idxh_v)

    def zero_srows():
        zv = jnp.zeros((L,), jnp.float32)

        def zrow(rr, carry):
            for off in (0, 16, 32, 48, 56):
                srows_v[rr, pl.ds(off, L)] = zv
            return carry
        lax.fori_loop(0, RCH, zrow, 0)

    def zero_acc():
        for q in range(NRCH):
            base = s * RPS + q * RCH
            pltpu.sync_copy(srows_v, acc_sh.at[pl.ds(base, RCH)])

    def scale_srows():
        def srow(r, carry):
            tail = srows_v[r, pl.ds(56, L)]
            invvec = jnp.where(tail > 0.0, 1.0 / tail, 0.0)
            vinv = jnp.full((L,), invvec[8], jnp.float32)
            for g in range(4):
                srows_v[r, pl.ds(g * L, L)] = srows_v[r, pl.ds(g * L, L)] * vinv
            srows_v[r, pl.ds(56, L)] = tail * vinv
            return carry
        lax.fori_loop(0, RCH, srow, 0)

    zero_srows()
    zero_acc()
    plsc.subcore_barrier()

    def run_pass(gather_from, idxg_v, idxs_v):
        def g_start(j, b, sm):
            pltpu.async_copy(gather_from.at[idxg_v.at[j]], grows_v.at[b], sm)

        def g_wait(j, b, sm):
            pltpu.make_async_copy(gather_from.at[idxg_v.at[j]],
                                  grows_v.at[b], sm).wait()

        def s_start(j, b, sm):
            pltpu.async_copy(grows_v.at[b], acc_sh.at[idxs_v.at[j]], sm, add=True)

        def s_wait(j, b, sm):
            pltpu.make_async_copy(grows_v.at[b],
                                  acc_sh.at[idxs_v.at[j]], sm).wait()

        g_start(0, 0, gsem0)
        g_start(1, 1, gsem1)

        def body(k, carry):
            j0 = 2 * k
            j1 = 2 * k + 1
            j2 = lax.rem(2 * k + 2, NCH)
            j3 = lax.rem(2 * k + 3, NCH)
            g_wait(j0, 0, gsem0)
            s_start(j0, 0, ssem0)
            g_wait(j1, 1, gsem1)
            s_start(j1, 1, ssem1)
            s_wait(j0, 0, ssem0)
            g_start(j2, 0, gsem0)
            s_wait(j1, 1, ssem1)
            g_start(j3, 1, gsem1)
            return carry
        lax.fori_loop(0, NCH // 2, body, 0)
        g_wait(0, 0, gsem0)
        g_wait(1, 1, gsem1)

    run_pass(xaug_hbm.at[c], idxn_v, idxh_v)
    plsc.subcore_barrier()

    for q in range(NRCH):
        base = s * RPS + q * RCH
        pltpu.sync_copy(acc_sh.at[pl.ds(base, RCH)], srows_v)
        scale_srows()
        pltpu.sync_copy(srows_v, he_hbm.at[c].at[pl.ds(base, RCH)])
    zero_srows()
    zero_acc()
    plsc.subcore_barrier()

    run_pass(he_hbm.at[c], idxh_v, idxn_v)
    plsc.subcore_barrier()

    for q in range(NRCH):
        base = s * RPS + q * RCH
        pltpu.sync_copy(acc_sh.at[pl.ds(base, RCH)], srows_v)
        scale_srows()
        pltpu.sync_copy(srows_v, out_hbm.at[c].at[pl.ds(base, RCH)])


def _make_calls():
    pre_x = pl.pallas_call(
        _pre_x_body,
        grid=(GRID,),
        in_specs=[
            pl.BlockSpec((RB, C), lambda i: (i, 0)),
            pl.BlockSpec((C, C), lambda i: (0, 0)),
        ],
        out_specs=pl.BlockSpec((NC, RB, W), lambda i: (0, i, 0)),
        out_shape=jax.ShapeDtypeStruct((NC, N_NODES, W), jnp.float32),
    )
    pre_t = pl.pallas_call(
        _pre_t_body,
        grid=(GRID,),
        in_specs=[
            pl.BlockSpec((RB, TEMB_C), lambda i: (i, 0)),
            pl.BlockSpec((TEMB_C, C), lambda i: (0, 0)),
            pl.BlockSpec((1, C), lambda i: (0, 0)),
        ],
        out_specs=pl.BlockSpec((RB, C), lambda i: (i, 0)),
        out_shape=jax.ShapeDtypeStruct((N_NODES, C), jnp.float32),
    )
    post = pl.pallas_call(
        _post_body,
        grid=(GRID,),
        in_specs=[
            pl.BlockSpec((NC, RB, W), lambda i: (0, i, 0)),
            pl.BlockSpec((RB, C), lambda i: (i, 0)),
            pl.BlockSpec((1, C), lambda i: (0, 0)),
            pl.BlockSpec((C, C), lambda i: (0, 0)),
            pl.BlockSpec((1, C), lambda i: (0, 0)),
        ],
        out_specs=pl.BlockSpec((RB, C), lambda i: (i, 0)),
        out_shape=jax.ShapeDtypeStruct((N_NODES, C), jnp.float32),
    )
    sc = pl.kernel(
        _sc_body,
        out_type=[
            jax.ShapeDtypeStruct((NC, N_PAD, W), jnp.float32),
            jax.ShapeDtypeStruct((NC, N_PAD, W), jnp.float32),
        ],
        mesh=plsc.VectorSubcoreMesh(
            core_axis_name="c", subcore_axis_name="s",
            num_cores=NC, num_subcores=NS,
        ),
        scratch_types=[
            pltpu.VMEM((NCH, CH), jnp.int32),
            pltpu.VMEM((NCH, CH), jnp.int32),
            pltpu.VMEM((2, CH, W), jnp.float32),
            pltpu.VMEM((RCH, W), jnp.float32),
            pltpu.VMEM_SHARED((N_PAD, W), jnp.float32),
            pltpu.SemaphoreType.DMA,
            pltpu.SemaphoreType.DMA,
            pltpu.SemaphoreType.DMA,
            pltpu.SemaphoreType.DMA,
        ],
        compiler_params=pltpu.CompilerParams(use_tc_tiling_on_sc=False),
    )
    return pre_x, pre_t, post, sc


def kernel(x, incidence_matrix, temb, W_conv, b_hconv, W_proj, b_proj, W_time, b_time):
    pre_x, pre_t, post, sc = _make_calls()
    node_idx = incidence_matrix[0]
    he_idx = incidence_matrix[1]
    xaug = pre_x(x, W_conv)
    t = pre_t(temb, W_time, b_time.reshape(1, C))
    idxn = node_idx.reshape(NS, NCH, CH)
    idxh = he_idx.reshape(NS, NCH, CH)
    outpad, _he = sc(xaug, idxn, idxh)
    h = post(outpad, t, b_hconv.reshape(1, C), W_proj, b_proj.reshape(1, C))
    return (h, jnp.zeros_like(x))


# speedup vs baseline: 21.8799x; 1.0008x over previous
import functools

import jax
import jax.numpy as jnp
from jax import lax
from jax.experimental import pallas as pl
from jax.experimental.pallas import tpu as pltpu
from jax.experimental.pallas import tpu_sc as plsc

N_NODES = 10000
N_HE = 10000
NNZ = 320000
C = 128
TEMB_C = 512
HALF = 64
W = 72
NC = 2
NS = 16
L = 16
EPS = NNZ // NS
CH = 125
NCH = EPS // CH
N_PAD = 10240
RPS = N_PAD // NS
RCH = 128
NRCH = RPS // RCH
RB = 1000
GRID = N_NODES // RB


def _pre_x_body(x_ref, wc_ref, xaug_ref):
    xt = jnp.dot(x_ref[...], wc_ref[...], preferred_element_type=jnp.float32)
    r = xt.shape[0]
    ones = jnp.ones((r, 1), jnp.float32)
    pad = jnp.zeros((r, W - HALF - 1), jnp.float32)
    h0 = jnp.concatenate([xt[:, :HALF], ones, pad], axis=1)
    h1 = jnp.concatenate([xt[:, HALF:], ones, pad], axis=1)
    xaug_ref[...] = jnp.stack([h0, h1], axis=0)


def _pre_t_body(temb_ref, wt_ref, bt_ref, t_ref):
    s = temb_ref[...]
    s = s * jax.nn.sigmoid(s)
    t_ref[...] = jnp.dot(s, wt_ref[...], preferred_element_type=jnp.float32) + bt_ref[...]


def _post_body(oa_ref, t_ref, bh_ref, wp_ref, bp_ref, h_ref):
    o = jnp.concatenate([oa_ref[0, :, :HALF], oa_ref[1, :, :HALF]], axis=1)
    o = o + bh_ref[...]
    hh = jnp.dot(o, wp_ref[...], preferred_element_type=jnp.float32)
    hh = hh + bp_ref[...] + t_ref[...]
    h_ref[...] = hh * jax.nn.sigmoid(hh)


def _sc_body(xaug_hbm, idxn_hbm, idxh_hbm,
             out_hbm, he_hbm,
             idxn_v, idxh_v, grows_v, srows_v, acc_sh, gsem0, gsem1, ssem0, ssem1):
    c = lax.axis_index("c")
    s = lax.axis_index("s")

    pltpu.sync_copy(idxn_hbm.at[s], idxn_v)
    pltpu.sync_copy(idxh_hbm.at[s], idxh_v)

    def zero_srows():
        zv = jnp.zeros((L,), jnp.float32)

        def zrow(rr, carry):
            for off in (0, 16, 32, 48, 56):
                srows_v[rr, pl.ds(off, L)] = zv
            return carry
        lax.fori_loop(0, RCH, zrow, 0)

    def zero_acc():
        for q in range(NRCH):
            base = s * RPS + q * RCH
            pltpu.sync_copy(srows_v, acc_sh.at[pl.ds(base, RCH)])

    def scale_srows():
        def srow(r, carry):
            tail = srows_v[r, pl.ds(56, L)]
            invvec = jnp.where(tail > 0.0, 1.0 / tail, 0.0)
            vinv = jnp.full((L,), invvec[8], jnp.float32)
            for g in range(4):
                srows_v[r, pl.ds(g * L, L)] = srows_v[r, pl.ds(g * L, L)] * vinv
            srows_v[r, pl.ds(56, L)] = tail * vinv
            return carry
        lax.fori_loop(0, RCH, srow, 0)

    zero_srows()
    zero_acc()
    plsc.subcore_barrier()

    def run_pass(gather_from, idxg_v, idxs_v):
        def g_start(j, b, sm):
            pltpu.async_copy(gather_from.at[idxg_v.at[j]], grows_v.at[b], sm)

        def g_wait(j, b, sm):
            pltpu.make_async_copy(gather_from.at[idxg_v.at[j]],
                                  grows_v.at[b], sm).wait()

        def s_start(j, b, sm):
            pltpu.async_copy(grows_v.at[b], acc_sh.at[idxs_v.at[j]], sm, add=True)

        def s_wait(j, b, sm):
            pltpu.make_async_copy(grows_v.at[b],
                                  acc_sh.at[idxs_v.at[j]], sm).wait()

        g_start(0, 0, gsem0)
        g_start(1, 1, gsem1)

        def body(k, carry):
            j0 = 2 * k
            j1 = 2 * k + 1
            j2 = lax.rem(2 * k + 2, NCH)
            j3 = lax.rem(2 * k + 3, NCH)
            g_wait(j0, 0, gsem0)
            s_start(j0, 0, ssem0)
            g_wait(j1, 1, gsem1)
            s_start(j1, 1, ssem1)
            s_wait(j0, 0, ssem0)
            g_start(j2, 0, gsem0)
            s_wait(j1, 1, ssem1)
            g_start(j3, 1, gsem1)
            return carry
        lax.fori_loop(0, NCH // 2, body, 0)
        g_wait(0, 0, gsem0)
        g_wait(1, 1, gsem1)

    run_pass(xaug_hbm.at[c], idxn_v, idxh_v)
    plsc.subcore_barrier()

    with jax.named_scope("ph2_scale_he"):
        for q in range(NRCH):
            base = s * RPS + q * RCH
            pltpu.sync_copy(acc_sh.at[pl.ds(base, RCH)], srows_v)
            scale_srows()
            pltpu.sync_copy(srows_v, he_hbm.at[c].at[pl.ds(base, RCH)])
        zero_srows()
        zero_acc()
        plsc.subcore_barrier()

    with jax.named_scope("ph3_pass2"):
        run_pass(he_hbm.at[c], idxh_v, idxn_v)
        plsc.subcore_barrier()

    with jax.named_scope("ph4_scale_out"):
        for q in range(NRCH):
            base = s * RPS + q * RCH
            pltpu.sync_copy(acc_sh.at[pl.ds(base, RCH)], srows_v)
            scale_srows()
            pltpu.sync_copy(srows_v, out_hbm.at[c].at[pl.ds(base, RCH)])


def _make_calls():
    pre_x = pl.pallas_call(
        _pre_x_body,
        grid=(GRID,),
        in_specs=[
            pl.BlockSpec((RB, C), lambda i: (i, 0)),
            pl.BlockSpec((C, C), lambda i: (0, 0)),
        ],
        out_specs=pl.BlockSpec((NC, RB, W), lambda i: (0, i, 0)),
        out_shape=jax.ShapeDtypeStruct((NC, N_NODES, W), jnp.float32),
    )
    pre_t = pl.pallas_call(
        _pre_t_body,
        grid=(GRID,),
        in_specs=[
            pl.BlockSpec((RB, TEMB_C), lambda i: (i, 0)),
            pl.BlockSpec((TEMB_C, C), lambda i: (0, 0)),
            pl.BlockSpec((1, C), lambda i: (0, 0)),
        ],
        out_specs=pl.BlockSpec((RB, C), lambda i: (i, 0)),
        out_shape=jax.ShapeDtypeStruct((N_NODES, C), jnp.float32),
    )
    post = pl.pallas_call(
        _post_body,
        grid=(GRID,),
        in_specs=[
            pl.BlockSpec((NC, RB, W), lambda i: (0, i, 0)),
            pl.BlockSpec((RB, C), lambda i: (i, 0)),
            pl.BlockSpec((1, C), lambda i: (0, 0)),
            pl.BlockSpec((C, C), lambda i: (0, 0)),
            pl.BlockSpec((1, C), lambda i: (0, 0)),
        ],
        out_specs=pl.BlockSpec((RB, C), lambda i: (i, 0)),
        out_shape=jax.ShapeDtypeStruct((N_NODES, C), jnp.float32),
    )
    sc = pl.kernel(
        _sc_body,
        out_type=[
            jax.ShapeDtypeStruct((NC, N_PAD, W), jnp.float32),
            jax.ShapeDtypeStruct((NC, N_PAD, W), jnp.float32),
        ],
        mesh=plsc.VectorSubcoreMesh(
            core_axis_name="c", subcore_axis_name="s",
            num_cores=NC, num_subcores=NS,
        ),
        scratch_types=[
            pltpu.VMEM((NCH, CH), jnp.int32),
            pltpu.VMEM((NCH, CH), jnp.int32),
            pltpu.VMEM((2, CH, W), jnp.float32),
            pltpu.VMEM((RCH, W), jnp.float32),
            pltpu.VMEM_SHARED((N_PAD, W), jnp.float32),
            pltpu.SemaphoreType.DMA,
            pltpu.SemaphoreType.DMA,
            pltpu.SemaphoreType.DMA,
            pltpu.SemaphoreType.DMA,
        ],
        compiler_params=pltpu.CompilerParams(use_tc_tiling_on_sc=False),
    )
    return pre_x, pre_t, post, sc


def kernel(x, incidence_matrix, temb, W_conv, b_hconv, W_proj, b_proj, W_time, b_time):
    pre_x, pre_t, post, sc = _make_calls()
    node_idx = incidence_matrix[0]
    he_idx = incidence_matrix[1]
    xaug = pre_x(x, W_conv)
    t = pre_t(temb, W_time, b_time.reshape(1, C))
    idxn = node_idx.reshape(NS, NCH, CH)
    idxh = he_idx.reshape(NS, NCH, CH)
    outpad, _he = sc(xaug, idxn, idxh)
    h = post(outpad, t, b_hconv.reshape(1, C), W_proj, b_proj.reshape(1, C))
    return (h, jnp.zeros_like(x))


# speedup vs baseline: 23.5609x; 1.0768x over previous
import functools

import jax
import jax.numpy as jnp
from jax import lax
from jax.experimental import pallas as pl
from jax.experimental.pallas import tpu as pltpu
from jax.experimental.pallas import tpu_sc as plsc

N_NODES = 10000
N_HE = 10000
NNZ = 320000
C = 128
TEMB_C = 512
HALF = 64
W = 72
NC = 2
NS = 16
L = 16
EPS = NNZ // NS
CH = 200
NCH = EPS // CH
N_PAD = 10240
RPS = N_PAD // NS
RCH = 128
NRCH = RPS // RCH
RB = 1000
GRID = N_NODES // RB


def _pre_x_body(x_ref, wc_ref, xaug_ref):
    xt = jnp.dot(x_ref[...], wc_ref[...], preferred_element_type=jnp.float32)
    r = xt.shape[0]
    ones = jnp.ones((r, 1), jnp.float32)
    pad = jnp.zeros((r, W - HALF - 1), jnp.float32)
    h0 = jnp.concatenate([xt[:, :HALF], ones, pad], axis=1)
    h1 = jnp.concatenate([xt[:, HALF:], ones, pad], axis=1)
    xaug_ref[...] = jnp.stack([h0, h1], axis=0)


def _pre_t_body(temb_ref, wt_ref, bt_ref, t_ref):
    s = temb_ref[...]
    s = s * jax.nn.sigmoid(s)
    t_ref[...] = jnp.dot(s, wt_ref[...], preferred_element_type=jnp.float32) + bt_ref[...]


def _post_body(oa_ref, t_ref, bh_ref, wp_ref, bp_ref, h_ref):
    o = jnp.concatenate([oa_ref[0, :, :HALF], oa_ref[1, :, :HALF]], axis=1)
    o = o + bh_ref[...]
    hh = jnp.dot(o, wp_ref[...], preferred_element_type=jnp.float32)
    hh = hh + bp_ref[...] + t_ref[...]
    h_ref[...] = hh * jax.nn.sigmoid(hh)


def _sc_body(xaug_hbm, idxn_hbm, idxh_hbm,
             out_hbm, he_hbm,
             idxn_v, idxh_v, grows_v, srows_v, acc_sh, gsem0, gsem1, ssem0, ssem1):
    c = lax.axis_index("c")
    s = lax.axis_index("s")

    pltpu.sync_copy(idxn_hbm.at[s], idxn_v)
    pltpu.sync_copy(idxh_hbm.at[s], idxh_v)

    def zero_srows():
        zv = jnp.zeros((L,), jnp.float32)

        def zrow(rr, carry):
            for off in (0, 16, 32, 48, 56):
                srows_v[rr, pl.ds(off, L)] = zv
            return carry
        lax.fori_loop(0, RCH, zrow, 0)

    def zero_acc():
        for q in range(NRCH):
            base = s * RPS + q * RCH
            pltpu.sync_copy(srows_v, acc_sh.at[pl.ds(base, RCH)])

    def scale_srows():
        def srow(r, carry):
            tail = srows_v[r, pl.ds(56, L)]
            invvec = jnp.where(tail > 0.0, 1.0 / tail, 0.0)
            vinv = jnp.full((L,), invvec[8], jnp.float32)
            for g in range(4):
                srows_v[r, pl.ds(g * L, L)] = srows_v[r, pl.ds(g * L, L)] * vinv
            srows_v[r, pl.ds(56, L)] = tail * vinv
            return carry
        lax.fori_loop(0, RCH, srow, 0)

    zero_srows()
    zero_acc()
    plsc.subcore_barrier()

    def run_pass(gather_from, idxg_v, idxs_v):
        def g_start(j, b, sm):
            pltpu.async_copy(gather_from.at[idxg_v.at[j]], grows_v.at[b], sm)

        def g_wait(j, b, sm):
            pltpu.make_async_copy(gather_from.at[idxg_v.at[j]],
                                  grows_v.at[b], sm).wait()

        def s_start(j, b, sm):
            pltpu.async_copy(grows_v.at[b], acc_sh.at[idxs_v.at[j]], sm, add=True)

        def s_wait(j, b, sm):
            pltpu.make_async_copy(grows_v.at[b],
                                  acc_sh.at[idxs_v.at[j]], sm).wait()

        g_start(0, 0, gsem0)
        g_start(1, 1, gsem1)

        def body(k, carry):
            j0 = 2 * k
            j1 = 2 * k + 1
            j2 = lax.rem(2 * k + 2, NCH)
            j3 = lax.rem(2 * k + 3, NCH)
            g_wait(j0, 0, gsem0)
            s_start(j0, 0, ssem0)
            g_wait(j1, 1, gsem1)
            s_start(j1, 1, ssem1)
            s_wait(j0, 0, ssem0)
            g_start(j2, 0, gsem0)
            s_wait(j1, 1, ssem1)
            g_start(j3, 1, gsem1)
            return carry
        lax.fori_loop(0, NCH // 2, body, 0)
        g_wait(0, 0, gsem0)
        g_wait(1, 1, gsem1)

    run_pass(xaug_hbm.at[c], idxn_v, idxh_v)
    plsc.subcore_barrier()

    with jax.named_scope("ph2_scale_he"):
        for q in range(NRCH):
            base = s * RPS + q * RCH
            pltpu.sync_copy(acc_sh.at[pl.ds(base, RCH)], srows_v)
            scale_srows()
            pltpu.sync_copy(srows_v, he_hbm.at[c].at[pl.ds(base, RCH)])
        zero_srows()
        zero_acc()
        plsc.subcore_barrier()

    with jax.named_scope("ph3_pass2"):
        run_pass(he_hbm.at[c], idxh_v, idxn_v)
        plsc.subcore_barrier()

    with jax.named_scope("ph4_scale_out"):
        for q in range(NRCH):
            base = s * RPS + q * RCH
            pltpu.sync_copy(acc_sh.at[pl.ds(base, RCH)], srows_v)
            scale_srows()
            pltpu.sync_copy(srows_v, out_hbm.at[c].at[pl.ds(base, RCH)])


def _make_calls():
    pre_x = pl.pallas_call(
        _pre_x_body,
        grid=(GRID,),
        in_specs=[
            pl.BlockSpec((RB, C), lambda i: (i, 0)),
            pl.BlockSpec((C, C), lambda i: (0, 0)),
        ],
        out_specs=pl.BlockSpec((NC, RB, W), lambda i: (0, i, 0)),
        out_shape=jax.ShapeDtypeStruct((NC, N_NODES, W), jnp.float32),
    )
    pre_t = pl.pallas_call(
        _pre_t_body,
        grid=(GRID,),
        in_specs=[
            pl.BlockSpec((RB, TEMB_C), lambda i: (i, 0)),
            pl.BlockSpec((TEMB_C, C), lambda i: (0, 0)),
            pl.BlockSpec((1, C), lambda i: (0, 0)),
        ],
        out_specs=pl.BlockSpec((RB, C), lambda i: (i, 0)),
        out_shape=jax.ShapeDtypeStruct((N_NODES, C), jnp.float32),
    )
    post = pl.pallas_call(
        _post_body,
        grid=(GRID,),
        in_specs=[
            pl.BlockSpec((NC, RB, W), lambda i: (0, i, 0)),
            pl.BlockSpec((RB, C), lambda i: (i, 0)),
            pl.BlockSpec((1, C), lambda i: (0, 0)),
            pl.BlockSpec((C, C), lambda i: (0, 0)),
            pl.BlockSpec((1, C), lambda i: (0, 0)),
        ],
        out_specs=pl.BlockSpec((RB, C), lambda i: (i, 0)),
        out_shape=jax.ShapeDtypeStruct((N_NODES, C), jnp.float32),
    )
    sc = pl.kernel(
        _sc_body,
        out_type=[
            jax.ShapeDtypeStruct((NC, N_PAD, W), jnp.float32),
            jax.ShapeDtypeStruct((NC, N_PAD, W), jnp.float32),
        ],
        mesh=plsc.VectorSubcoreMesh(
            core_axis_name="c", subcore_axis_name="s",
            num_cores=NC, num_subcores=NS,
        ),
        scratch_types=[
            pltpu.VMEM((NCH, CH), jnp.int32),
            pltpu.VMEM((NCH, CH), jnp.int32),
            pltpu.VMEM((2, CH, W), jnp.float32),
            pltpu.VMEM((RCH, W), jnp.float32),
            pltpu.VMEM_SHARED((N_PAD, W), jnp.float32),
            pltpu.SemaphoreType.DMA,
            pltpu.SemaphoreType.DMA,
            pltpu.SemaphoreType.DMA,
            pltpu.SemaphoreType.DMA,
        ],
        compiler_params=pltpu.CompilerParams(use_tc_tiling_on_sc=False),
    )
    return pre_x, pre_t, post, sc


def kernel(x, incidence_matrix, temb, W_conv, b_hconv, W_proj, b_proj, W_time, b_time):
    pre_x, pre_t, post, sc = _make_calls()
    node_idx = incidence_matrix[0]
    he_idx = incidence_matrix[1]
    xaug = pre_x(x, W_conv)
    t = pre_t(temb, W_time, b_time.reshape(1, C))
    idxn = node_idx.reshape(NS, NCH, CH)
    idxh = he_idx.reshape(NS, NCH, CH)
    outpad, _he = sc(xaug, idxn, idxh)
    h = post(outpad, t, b_hconv.reshape(1, C), W_proj, b_proj.reshape(1, C))
    return (h, jnp.zeros_like(x))


# speedup vs baseline: 23.7116x; 1.0064x over previous
import functools

import jax
import jax.numpy as jnp
from jax import lax
from jax.experimental import pallas as pl
from jax.experimental.pallas import tpu as pltpu
from jax.experimental.pallas import tpu_sc as plsc

N_NODES = 10000
N_HE = 10000
NNZ = 320000
C = 128
TEMB_C = 512
HALF = 64
W = 72
NC = 2
NS = 16
L = 16
EPS = NNZ // NS
CH = 250
NCH = EPS // CH
N_PAD = 10240
RPS = N_PAD // NS
RCH = 128
NRCH = RPS // RCH
RB = 1000
GRID = N_NODES // RB


def _pre_x_body(x_ref, wc_ref, xaug_ref):
    xt = jnp.dot(x_ref[...], wc_ref[...], preferred_element_type=jnp.float32)
    r = xt.shape[0]
    ones = jnp.ones((r, 1), jnp.float32)
    pad = jnp.zeros((r, W - HALF - 1), jnp.float32)
    h0 = jnp.concatenate([xt[:, :HALF], ones, pad], axis=1)
    h1 = jnp.concatenate([xt[:, HALF:], ones, pad], axis=1)
    xaug_ref[...] = jnp.stack([h0, h1], axis=0)


def _pre_t_body(temb_ref, wt_ref, bt_ref, t_ref):
    s = temb_ref[...]
    s = s * jax.nn.sigmoid(s)
    t_ref[...] = jnp.dot(s, wt_ref[...], preferred_element_type=jnp.float32) + bt_ref[...]


def _post_body(oa_ref, t_ref, bh_ref, wp_ref, bp_ref, h_ref):
    o = jnp.concatenate([oa_ref[0, :, :HALF], oa_ref[1, :, :HALF]], axis=1)
    o = o + bh_ref[...]
    hh = jnp.dot(o, wp_ref[...], preferred_element_type=jnp.float32)
    hh = hh + bp_ref[...] + t_ref[...]
    h_ref[...] = hh * jax.nn.sigmoid(hh)


def _sc_body(xaug_hbm, idxn_hbm, idxh_hbm,
             out_hbm, he_hbm,
             idxn_v, idxh_v, grows_v, acc_sh, gsem0, gsem1, ssem0, ssem1):
    srows_v = grows_v.at[0].at[pl.ds(0, RCH)]
    c = lax.axis_index("c")
    s = lax.axis_index("s")

    pltpu.sync_copy(idxn_hbm.at[s], idxn_v)
    pltpu.sync_copy(idxh_hbm.at[s], idxh_v)

    def zero_srows():
        zv = jnp.zeros((L,), jnp.float32)

        def zrow(rr, carry):
            for off in (0, 16, 32, 48, 56):
                srows_v[rr, pl.ds(off, L)] = zv
            return carry
        lax.fori_loop(0, RCH, zrow, 0)

    def zero_acc():
        for q in range(NRCH):
            base = s * RPS + q * RCH
            pltpu.sync_copy(srows_v, acc_sh.at[pl.ds(base, RCH)])

    def scale_srows():
        def srow(r, carry):
            tail = srows_v[r, pl.ds(56, L)]
            invvec = jnp.where(tail > 0.0, 1.0 / tail, 0.0)
            vinv = jnp.full((L,), invvec[8], jnp.float32)
            for g in range(4):
                srows_v[r, pl.ds(g * L, L)] = srows_v[r, pl.ds(g * L, L)] * vinv
            srows_v[r, pl.ds(56, L)] = tail * vinv
            return carry
        lax.fori_loop(0, RCH, srow, 0)

    zero_srows()
    zero_acc()
    plsc.subcore_barrier()

    def run_pass(gather_from, idxg_v, idxs_v):
        def g_start(j, b, sm):
            pltpu.async_copy(gather_from.at[idxg_v.at[j]], grows_v.at[b], sm)

        def g_wait(j, b, sm):
            pltpu.make_async_copy(gather_from.at[idxg_v.at[j]],
                                  grows_v.at[b], sm).wait()

        def s_start(j, b, sm):
            pltpu.async_copy(grows_v.at[b], acc_sh.at[idxs_v.at[j]], sm, add=True)

        def s_wait(j, b, sm):
            pltpu.make_async_copy(grows_v.at[b],
                                  acc_sh.at[idxs_v.at[j]], sm).wait()

        g_start(0, 0, gsem0)
        g_start(1, 1, gsem1)

        def body(k, carry):
            j0 = 2 * k
            j1 = 2 * k + 1
            j2 = lax.rem(2 * k + 2, NCH)
            j3 = lax.rem(2 * k + 3, NCH)
            g_wait(j0, 0, gsem0)
            s_start(j0, 0, ssem0)
            g_wait(j1, 1, gsem1)
            s_start(j1, 1, ssem1)
            s_wait(j0, 0, ssem0)
            g_start(j2, 0, gsem0)
            s_wait(j1, 1, ssem1)
            g_start(j3, 1, gsem1)
            return carry
        lax.fori_loop(0, NCH // 2, body, 0)
        g_wait(0, 0, gsem0)
        g_wait(1, 1, gsem1)

    run_pass(xaug_hbm.at[c], idxn_v, idxh_v)
    plsc.subcore_barrier()

    with jax.named_scope("ph2_scale_he"):
        for q in range(NRCH):
            base = s * RPS + q * RCH
            pltpu.sync_copy(acc_sh.at[pl.ds(base, RCH)], srows_v)
            scale_srows()
            pltpu.sync_copy(srows_v, he_hbm.at[c].at[pl.ds(base, RCH)])
        zero_srows()
        zero_acc()
        plsc.subcore_barrier()

    with jax.named_scope("ph3_pass2"):
        run_pass(he_hbm.at[c], idxh_v, idxn_v)
        plsc.subcore_barrier()

    with jax.named_scope("ph4_scale_out"):
        for q in range(NRCH):
            base = s * RPS + q * RCH
            pltpu.sync_copy(acc_sh.at[pl.ds(base, RCH)], srows_v)
            scale_srows()
            pltpu.sync_copy(srows_v, out_hbm.at[c].at[pl.ds(base, RCH)])


def _make_calls():
    pre_x = pl.pallas_call(
        _pre_x_body,
        grid=(GRID,),
        in_specs=[
            pl.BlockSpec((RB, C), lambda i: (i, 0)),
            pl.BlockSpec((C, C), lambda i: (0, 0)),
        ],
        out_specs=pl.BlockSpec((NC, RB, W), lambda i: (0, i, 0)),
        out_shape=jax.ShapeDtypeStruct((NC, N_NODES, W), jnp.float32),
    )
    pre_t = pl.pallas_call(
        _pre_t_body,
        grid=(GRID,),
        in_specs=[
            pl.BlockSpec((RB, TEMB_C), lambda i: (i, 0)),
            pl.BlockSpec((TEMB_C, C), lambda i: (0, 0)),
            pl.BlockSpec((1, C), lambda i: (0, 0)),
        ],
        out_specs=pl.BlockSpec((RB, C), lambda i: (i, 0)),
        out_shape=jax.ShapeDtypeStruct((N_NODES, C), jnp.float32),
    )
    post = pl.pallas_call(
        _post_body,
        grid=(GRID,),
        in_specs=[
            pl.BlockSpec((NC, RB, W), lambda i: (0, i, 0)),
            pl.BlockSpec((RB, C), lambda i: (i, 0)),
            pl.BlockSpec((1, C), lambda i: (0, 0)),
            pl.BlockSpec((C, C), lambda i: (0, 0)),
            pl.BlockSpec((1, C), lambda i: (0, 0)),
        ],
        out_specs=pl.BlockSpec((RB, C), lambda i: (i, 0)),
        out_shape=jax.ShapeDtypeStruct((N_NODES, C), jnp.float32),
    )
    sc = pl.kernel(
        _sc_body,
        out_type=[
            jax.ShapeDtypeStruct((NC, N_PAD, W), jnp.float32),
            jax.ShapeDtypeStruct((NC, N_PAD, W), jnp.float32),
        ],
        mesh=plsc.VectorSubcoreMesh(
            core_axis_name="c", subcore_axis_name="s",
            num_cores=NC, num_subcores=NS,
        ),
        scratch_types=[
            pltpu.VMEM((NCH, CH), jnp.int32),
            pltpu.VMEM((NCH, CH), jnp.int32),
            pltpu.VMEM((2, CH, W), jnp.float32),
            pltpu.VMEM_SHARED((N_PAD, W), jnp.float32),
            pltpu.SemaphoreType.DMA,
            pltpu.SemaphoreType.DMA,
            pltpu.SemaphoreType.DMA,
            pltpu.SemaphoreType.DMA,
        ],
        compiler_params=pltpu.CompilerParams(use_tc_tiling_on_sc=False),
    )
    return pre_x, pre_t, post, sc


def kernel(x, incidence_matrix, temb, W_conv, b_hconv, W_proj, b_proj, W_time, b_time):
    pre_x, pre_t, post, sc = _make_calls()
    node_idx = incidence_matrix[0]
    he_idx = incidence_matrix[1]
    xaug = pre_x(x, W_conv)
    t = pre_t(temb, W_time, b_time.reshape(1, C))
    idxn = node_idx.reshape(NS, NCH, CH)
    idxh = he_idx.reshape(NS, NCH, CH)
    outpad, _he = sc(xaug, idxn, idxh)
    h = post(outpad, t, b_hconv.reshape(1, C), W_proj, b_proj.reshape(1, C))
    return (h, jnp.zeros_like(x))
